# baseline (speedup 1.0000x reference)
import functools

import jax
import jax.numpy as jnp
from jax import lax
from jax.experimental import pallas as pl
from jax.experimental.pallas import tpu as pltpu
from jax.experimental.pallas import tpu_sc as plsc

VOCAB = 1000
DIM = 32
B = 16384
H = 50
NC, NS = 2, 16
NW = NC * NS
B_PER_W = B // NW
NB = 16
MACROS = B_PER_W // NB


def _make_sc_gather():
    mesh = plsc.VectorSubcoreMesh(core_axis_name="c", subcore_axis_name="s")

    @functools.partial(
        pl.kernel,
        mesh=mesh,
        compiler_params=pltpu.CompilerParams(use_tc_tiling_on_sc=False),
        out_type=jax.ShapeDtypeStruct((B * H, DIM), jnp.float32),
        scratch_types=[
            pltpu.VMEM((B_PER_W, H), jnp.int32),
            pltpu.VMEM((NB * H, DIM), jnp.float32),
            pltpu.VMEM((NB * H, DIM), jnp.float32),
            pltpu.SemaphoreType.DMA,
            pltpu.SemaphoreType.DMA,
        ],
    )
    def k(table_hbm, idx_hbm, out_hbm, idx_v, rows0_v, rows1_v, sem_g, sem_o):
        wid = lax.axis_index("s") * NC + lax.axis_index("c")
        b0 = wid * B_PER_W
        pltpu.sync_copy(idx_hbm.at[pl.ds(b0, B_PER_W)], idx_v)

        def gather_into(m, buf):
            r0 = m * NB
            cps = [
                pltpu.async_copy(
                    table_hbm.at[idx_v.at[r0 + i]],
                    buf.at[pl.ds(i * H, H)],
                    sem_g,
                )
                for i in range(NB)
            ]
            for cp in cps:
                cp.wait()

        def start_write(m, buf):
            pltpu.async_copy(
                buf, out_hbm.at[pl.ds((b0 + m * NB) * H, NB * H)], sem_o
            )

        def wait_write(buf):
            pltpu.make_async_copy(
                buf, out_hbm.at[pl.ds(b0 * H, NB * H)], sem_o
            ).wait()

        gather_into(0, rows0_v)
        start_write(0, rows0_v)
        gather_into(1, rows1_v)
        start_write(1, rows1_v)

        def macro(mm, carry):
            m = 2 * mm
            wait_write(rows0_v)
            gather_into(m, rows0_v)
            start_write(m, rows0_v)
            wait_write(rows1_v)
            gather_into(m + 1, rows1_v)
            start_write(m + 1, rows1_v)
            return carry

        lax.fori_loop(1, MACROS // 2, macro, 0, unroll=False)

        wait_write(rows0_v)
        wait_write(rows1_v)

    return k


_sc_gather = _make_sc_gather()


def kernel(batch, weight):
    flat = _sc_gather(weight, batch.astype(jnp.int32))
    return flat.reshape(B, H, DIM)


# speedup vs baseline: 2.0018x; 2.0018x over previous
import functools

import jax
import jax.numpy as jnp
from jax import lax
from jax.experimental import pallas as pl
from jax.experimental.pallas import tpu as pltpu
from jax.experimental.pallas import tpu_sc as plsc

VOCAB = 1000
DIM = 32
B = 16384
H = 50
NC, NS = 2, 16
NW = NC * NS
B_PER_W = B // NW
NB = 16
MACROS = B_PER_W // NB


def _make_sc_gather():
    mesh = plsc.VectorSubcoreMesh(core_axis_name="c", subcore_axis_name="s")

    @functools.partial(
        pl.kernel,
        mesh=mesh,
        compiler_params=pltpu.CompilerParams(use_tc_tiling_on_sc=False),
        out_type=jax.ShapeDtypeStruct((B, H, DIM), jnp.float32),
        scratch_types=[
            pltpu.VMEM((B_PER_W, H), jnp.int32),
            pltpu.VMEM((NB, H, DIM), jnp.float32),
            pltpu.VMEM((NB, H, DIM), jnp.float32),
            pltpu.SemaphoreType.DMA,
            pltpu.SemaphoreType.DMA,
        ],
    )
    def k(table_hbm, idx_hbm, out_hbm, idx_v, rows0_v, rows1_v, sem_g, sem_o):
        wid = lax.axis_index("s") * NC + lax.axis_index("c")
        b0 = wid * B_PER_W
        pltpu.sync_copy(idx_hbm.at[pl.ds(b0, B_PER_W)], idx_v)

        def gather_into(m, buf):
            r0 = m * NB
            cps = [
                pltpu.async_copy(
                    table_hbm.at[idx_v.at[r0 + i]], buf.at[i], sem_g
                )
                for i in range(NB)
            ]
            for cp in cps:
                cp.wait()

        def start_write(m, buf):
            pltpu.async_copy(
                buf, out_hbm.at[pl.ds(b0 + m * NB, NB)], sem_o
            )

        def wait_write(buf):
            pltpu.make_async_copy(
                buf, out_hbm.at[pl.ds(b0, NB)], sem_o
            ).wait()

        gather_into(0, rows0_v)
        start_write(0, rows0_v)
        gather_into(1, rows1_v)
        start_write(1, rows1_v)

        def macro(mm, carry):
            m = 2 * mm
            wait_write(rows0_v)
            gather_into(m, rows0_v)
            start_write(m, rows0_v)
            wait_write(rows1_v)
            gather_into(m + 1, rows1_v)
            start_write(m + 1, rows1_v)
            return carry

        lax.fori_loop(1, MACROS // 2, macro, 0, unroll=False)

        wait_write(rows0_v)
        wait_write(rows1_v)

    return k


_sc_gather = _make_sc_gather()


def kernel(batch, weight):
    return _sc_gather(weight, batch.astype(jnp.int32))
